# Initial kernel scaffold; baseline (speedup 1.0000x reference)
import jax
import jax.numpy as jnp
from jax.experimental import pallas as pl


def kernel(x, edge_index, W1l, b1l, W1r, W2l, b2l, W2r):
    raise NotImplementedError("write your pallas kernel here")


# speedup vs baseline: 4.8439x; 4.8439x over previous
import functools

import jax
import jax.numpy as jnp
from jax import lax
from jax.experimental import pallas as pl
from jax.experimental.pallas import tpu as pltpu
from jax.experimental.pallas import tpu_sc as plsc

_NC = 2
_NS = 16
_CHUNK = 80


@functools.partial(jax.jit, static_argnums=(0, 1, 2))
def _sc_agg_parts(n, d, e, x, src, dst, zeros):
    nw = _NC * _NS
    ew = e // nw
    steps = ew // _CHUNK
    rpt = (n // _NS) // 8 * 8
    rem = n - rpt * _NS

    mesh = plsc.VectorSubcoreMesh(core_axis_name="c", subcore_axis_name="s")

    @functools.partial(
        pl.kernel,
        mesh=mesh,
        out_type=jax.ShapeDtypeStruct((_NC, n, d), jnp.float32),
        scratch_types=[
            pltpu.VMEM((_CHUNK,), jnp.int32),
            pltpu.VMEM((_CHUNK,), jnp.int32),
            pltpu.VMEM((_CHUNK, d), jnp.float32),
            pltpu.SemaphoreType.DMA,
            pltpu.VMEM_SHARED((n, d), jnp.float32),
        ],
    )
    def k(x_hbm, src_hbm, dst_hbm, zero_hbm, out_hbm,
          src_v, dst_v, rows_v, sem, accum):
        cid = lax.axis_index("c")
        sid = lax.axis_index("s")
        wid = cid * _NS + sid

        pltpu.sync_copy(zero_hbm.at[pl.ds(sid * rpt, rpt)],
                        accum.at[pl.ds(sid * rpt, rpt)])
        if rem:
            @pl.when(sid == 0)
            def _():
                pltpu.sync_copy(zero_hbm.at[pl.ds(rpt * _NS, rem)],
                                accum.at[pl.ds(rpt * _NS, rem)])
        plsc.subcore_barrier()

        base = wid * ew

        def step(i, _):
            off = base + i * _CHUNK
            pltpu.sync_copy(src_hbm.at[pl.ds(off, _CHUNK)], src_v)
            pltpu.sync_copy(dst_hbm.at[pl.ds(off, _CHUNK)], dst_v)
            pltpu.async_copy(x_hbm.at[src_v], rows_v, sem).wait()
            pltpu.sync_copy(rows_v, accum.at[dst_v], add=True)
            return 0

        lax.fori_loop(0, steps, step, 0)
        plsc.subcore_barrier()

        pltpu.sync_copy(accum.at[pl.ds(sid * rpt, rpt)],
                        out_hbm.at[cid, pl.ds(sid * rpt, rpt)])
        if rem:
            @pl.when(sid == 0)
            def _():
                pltpu.sync_copy(accum.at[pl.ds(rpt * _NS, rem)],
                                out_hbm.at[cid, pl.ds(rpt * _NS, rem)])

    return k(x, src, dst, zeros)


def _tc_dense_body(p0_ref, p1_ref, x_ref, w1l_ref, b1_ref, w1r_ref,
                   w2l_ref, b2_ref, w2r_ref, loc_ref, scale_ref):
    agg = p0_ref[...] + p1_ref[...]
    xb = x_ref[...]
    h1 = (jnp.dot(agg, w1l_ref[...], preferred_element_type=jnp.float32)
          + jnp.dot(xb, w1r_ref[...], preferred_element_type=jnp.float32)
          + b1_ref[...])
    loc_ref[...] = jnp.clip(h1, -100.0, 100.0)
    h2 = (jnp.dot(agg, w2l_ref[...], preferred_element_type=jnp.float32)
          + jnp.dot(xb, w2r_ref[...], preferred_element_type=jnp.float32)
          + b2_ref[...])
    sp = jnp.maximum(h2, 0.0) + jnp.log1p(jnp.exp(-jnp.abs(h2)))
    scale_ref[...] = jnp.minimum(sp + 0.001, 100.0)


def _tc_dense(p0, p1, x, w1lT, b1, w1rT, w2lT, b2, w2rT):
    n, d = x.shape
    blk = 1000
    grid = (n // blk,)
    row_spec = pl.BlockSpec((blk, d), lambda i: (i, 0))
    w_spec = pl.BlockSpec((d, d), lambda i: (0, 0))
    b_spec = pl.BlockSpec((1, d), lambda i: (0, 0))
    return pl.pallas_call(
        _tc_dense_body,
        grid=grid,
        in_specs=[row_spec, row_spec, row_spec,
                  w_spec, b_spec, w_spec, w_spec, b_spec, w_spec],
        out_specs=[row_spec, row_spec],
        out_shape=[jax.ShapeDtypeStruct((n, d), jnp.float32),
                   jax.ShapeDtypeStruct((n, d), jnp.float32)],
    )(p0, p1, x, w1lT, b1, w1rT, w2lT, b2, w2rT)


def kernel(x, edge_index, W1l, b1l, W1r, W2l, b2l, W2r):
    n, d = x.shape
    e = edge_index.shape[1]
    src = edge_index[0]
    dst = edge_index[1]
    zeros = jnp.zeros((n, d), jnp.float32)
    parts = _sc_agg_parts(n, d, e, x, src, dst, zeros)
    loc, scale = _tc_dense(parts[0], parts[1], x,
                           W1l.T, b1l.reshape(1, d), W1r.T,
                           W2l.T, b2l.reshape(1, d), W2r.T)
    return (loc, scale)


# speedup vs baseline: 10.8479x; 2.2395x over previous
import functools

import jax
import jax.numpy as jnp
from jax import lax
from jax.experimental import pallas as pl
from jax.experimental.pallas import tpu as pltpu
from jax.experimental.pallas import tpu_sc as plsc

_NC = 2
_NS = 16
_C = 128
_PAD_ROWS = 16


@functools.partial(jax.jit, static_argnums=(0, 1, 2))
def _sc_agg_parts(n, d, steps, x, src2, dst2, zeros):
    n_acc = n + _PAD_ROWS
    rpt = (n // _NS) // 8 * 8
    rem = n - rpt * _NS

    mesh = plsc.VectorSubcoreMesh(core_axis_name="c", subcore_axis_name="s")

    @functools.partial(
        pl.kernel,
        mesh=mesh,
        out_type=jax.ShapeDtypeStruct((_NC, n, d), jnp.float32),
        scratch_types=[
            [pltpu.VMEM((_C,), jnp.int32)] * 4,
            [pltpu.VMEM((_C,), jnp.int32)] * 4,
            [pltpu.VMEM((_C, d), jnp.float32)] * 2,
            [pltpu.SemaphoreType.DMA] * 4,
            [pltpu.SemaphoreType.DMA] * 2,
            pltpu.VMEM_SHARED((n_acc, d), jnp.float32),
        ],
    )
    def k(x_hbm, src_hbm, dst_hbm, zero_hbm, out_hbm,
          sb, db, rows, si, sr, accum):
        cid = lax.axis_index("c")
        sid = lax.axis_index("s")
        wid = cid * _NS + sid

        pltpu.sync_copy(zero_hbm.at[pl.ds(sid * rpt, rpt)],
                        accum.at[pl.ds(sid * rpt, rpt)])
        if rem:
            @pl.when(sid == 0)
            def _():
                pltpu.sync_copy(zero_hbm.at[pl.ds(rpt * _NS, rem)],
                                accum.at[pl.ds(rpt * _NS, rem)])

        ebase = wid * steps * _C

        def idx_start(c, k_):
            off = ebase + c * _C
            pltpu.async_copy(src_hbm.at[pl.ds(off, _C)], sb[k_], si[k_])
            pltpu.async_copy(dst_hbm.at[pl.ds(off, _C)], db[k_], si[k_])

        def idx_wait(c, k_):
            off = ebase + c * _C
            pltpu.make_async_copy(src_hbm.at[pl.ds(off, _C)], sb[k_],
                                  si[k_]).wait()
            pltpu.make_async_copy(dst_hbm.at[pl.ds(off, _C)], db[k_],
                                  si[k_]).wait()

        def gather_start(k_, r_):
            pltpu.async_copy(x_hbm.at[sb[k_]], rows[r_], sr[r_])

        def gather_wait(k_, r_):
            pltpu.make_async_copy(x_hbm.at[sb[k_]], rows[r_], sr[r_]).wait()

        def scat(k_, r_):
            pltpu.sync_copy(rows[r_], accum.at[db[k_]], add=True)

        for k_ in range(4):
            idx_start(k_, k_)
        idx_wait(0, 0)
        gather_start(0, 0)
        plsc.subcore_barrier()

        def quad(j, _):
            c = 4 * j
            for k_ in range(4):
                idx_wait(c + k_ + 1, (k_ + 1) % 4)
                gather_start((k_ + 1) % 4, (k_ + 1) % 2)
                gather_wait(k_ % 4, k_ % 2)
                scat(k_ % 4, k_ % 2)
                idx_start(c + k_ + 4, k_)
            return 0

        lax.fori_loop(0, steps // 4 - 1, quad, 0)

        for k_ in range(4):
            if k_ < 3:
                idx_wait(steps - 3 + k_, (k_ + 1) % 4)
                gather_start((k_ + 1) % 4, (k_ + 1) % 2)
            gather_wait(k_ % 4, k_ % 2)
            scat(k_ % 4, k_ % 2)

        plsc.subcore_barrier()

        pltpu.sync_copy(accum.at[pl.ds(sid * rpt, rpt)],
                        out_hbm.at[cid, pl.ds(sid * rpt, rpt)])
        if rem:
            @pl.when(sid == 0)
            def _():
                pltpu.sync_copy(accum.at[pl.ds(rpt * _NS, rem)],
                                out_hbm.at[cid, pl.ds(rpt * _NS, rem)])

    return k(x, src2, dst2, zeros)


def _tc_dense_body(p0_ref, p1_ref, x_ref, w1l_ref, b1_ref, w1r_ref,
                   w2l_ref, b2_ref, w2r_ref, loc_ref, scale_ref):
    agg = p0_ref[...] + p1_ref[...]
    xb = x_ref[...]
    h1 = (jnp.dot(agg, w1l_ref[...], preferred_element_type=jnp.float32)
          + jnp.dot(xb, w1r_ref[...], preferred_element_type=jnp.float32)
          + b1_ref[...])
    loc_ref[...] = jnp.clip(h1, -100.0, 100.0)
    h2 = (jnp.dot(agg, w2l_ref[...], preferred_element_type=jnp.float32)
          + jnp.dot(xb, w2r_ref[...], preferred_element_type=jnp.float32)
          + b2_ref[...])
    sp = jnp.maximum(h2, 0.0) + jnp.log1p(jnp.exp(-jnp.abs(h2)))
    scale_ref[...] = jnp.minimum(sp + 0.001, 100.0)


def _tc_dense(p0, p1, x, w1lT, b1, w1rT, w2lT, b2, w2rT):
    n, d = x.shape
    blk = 1000
    grid = (n // blk,)
    row_spec = pl.BlockSpec((blk, d), lambda i: (i, 0))
    w_spec = pl.BlockSpec((d, d), lambda i: (0, 0))
    b_spec = pl.BlockSpec((1, d), lambda i: (0, 0))
    return pl.pallas_call(
        _tc_dense_body,
        grid=grid,
        in_specs=[row_spec, row_spec, row_spec,
                  w_spec, b_spec, w_spec, w_spec, b_spec, w_spec],
        out_specs=[row_spec, row_spec],
        out_shape=[jax.ShapeDtypeStruct((n, d), jnp.float32),
                   jax.ShapeDtypeStruct((n, d), jnp.float32)],
    )(p0, p1, x, w1lT, b1, w1rT, w2lT, b2, w2rT)


def kernel(x, edge_index, W1l, b1l, W1r, W2l, b2l, W2r):
    n, d = x.shape
    e = edge_index.shape[1]
    nw = _NC * _NS
    steps = -(-e // (_C * nw))
    steps = (steps + 3) // 4 * 4
    e_pad = steps * _C * nw

    src = edge_index[0]
    dst = edge_index[1]
    if e_pad > e:
        pad = e_pad - e
        src_pad = jnp.arange(pad, dtype=jnp.int32) % n
        dst_pad = n + jnp.arange(pad, dtype=jnp.int32) % _PAD_ROWS
        src = jnp.concatenate([src, src_pad])
        dst = jnp.concatenate([dst, dst_pad])
    src2 = src
    dst2 = dst
    zeros = jnp.zeros((n, d), jnp.float32)
    parts = _sc_agg_parts(n, d, steps, x, src2, dst2, zeros)
    loc, scale = _tc_dense(parts[0], parts[1], x,
                           W1l.T, b1l.reshape(1, d), W1r.T,
                           W2l.T, b2l.reshape(1, d), W2r.T)
    return (loc, scale)


# speedup vs baseline: 11.1850x; 1.0311x over previous
import functools

import jax
import jax.numpy as jnp
from jax import lax
from jax.experimental import pallas as pl
from jax.experimental.pallas import tpu as pltpu
from jax.experimental.pallas import tpu_sc as plsc

_NC = 2
_NS = 16
_C = 128


@functools.partial(jax.jit, static_argnums=(0, 1, 2, 3))
def _sc_agg_parts(n, d, steps, n_extra, x, src, dst):
    nw = _NC * _NS
    rpt = (n // _NS) // 8 * 8
    rem = n - rpt * _NS
    full = rpt // _C
    part = rpt - full * _C

    mesh = plsc.VectorSubcoreMesh(core_axis_name="c", subcore_axis_name="s")

    @functools.partial(
        pl.kernel,
        mesh=mesh,
        out_type=jax.ShapeDtypeStruct((_NC, n, d), jnp.float32),
        scratch_types=[
            [pltpu.VMEM((_C,), jnp.int32)] * 4,
            [pltpu.VMEM((_C,), jnp.int32)] * 4,
            [pltpu.VMEM((_C, d), jnp.float32)] * 2,
            [pltpu.SemaphoreType.DMA] * 4,
            [pltpu.SemaphoreType.DMA] * 2,
            pltpu.VMEM_SHARED((n, d), jnp.float32),
        ],
    )
    def k(x_hbm, src_hbm, dst_hbm, out_hbm, sb, db, rows, si, ss, accum):
        cid = lax.axis_index("c")
        sid = lax.axis_index("s")
        wid = cid * _NS + sid
        ebase = wid * steps * _C

        def idx_start(c, k_):
            off = ebase + c * _C
            pltpu.async_copy(src_hbm.at[pl.ds(off, _C)], sb[k_], si[k_])
            pltpu.async_copy(dst_hbm.at[pl.ds(off, _C)], db[k_], si[k_])

        def idx_wait(c, k_):
            off = ebase + c * _C
            pltpu.make_async_copy(src_hbm.at[pl.ds(off, _C)], sb[k_],
                                  si[k_]).wait()
            pltpu.make_async_copy(dst_hbm.at[pl.ds(off, _C)], db[k_],
                                  si[k_]).wait()

        def gather_start(k_, r_):
            pltpu.async_copy(x_hbm.at[sb[k_]], rows[r_], ss[r_])

        def gather_wait(k_, r_):
            pltpu.make_async_copy(x_hbm.at[sb[k_]], rows[r_], ss[r_]).wait()

        def scat_start(k_, r_):
            pltpu.async_copy(rows[r_], accum.at[db[k_]], ss[r_], add=True)

        def scat_wait(k_, r_):
            pltpu.make_async_copy(rows[r_], accum.at[db[k_]], ss[r_]).wait()

        for k_ in range(4):
            idx_start(k_, k_)

        zv = jnp.zeros((16,), jnp.float32)

        def zrow(r, _):
            for cc in range(d // 16):
                rows[0][r, pl.ds(cc * 16, 16)] = zv
            return 0

        lax.fori_loop(0, _C, zrow, 0)
        zbase = sid * rpt
        for b in range(full):
            pltpu.sync_copy(rows[0], accum.at[pl.ds(zbase + b * _C, _C)])
        if part:
            pltpu.sync_copy(rows[0].at[pl.ds(0, part)],
                            accum.at[pl.ds(zbase + full * _C, part)])
        if rem:
            @pl.when(sid == 0)
            def _():
                pltpu.sync_copy(rows[0].at[pl.ds(0, rem)],
                                accum.at[pl.ds(rpt * _NS, rem)])

        idx_wait(0, 0)
        gather_start(0, 0)
        plsc.subcore_barrier()

        def slot(q, k_, do_scat_wait=True, do_refill=True, do_next=True):
            if do_next:
                idx_wait(q + 1, (k_ + 1) % 4)
            if do_scat_wait:
                scat_wait((k_ - 1) % 4, (k_ + 1) % 2)
            if do_next:
                gather_start((k_ + 1) % 4, (k_ + 1) % 2)
            if do_refill:
                idx_start(q + 3, (k_ - 1) % 4)
            gather_wait(k_ % 4, k_ % 2)
            scat_start(k_ % 4, k_ % 2)

        slot(0, 0, do_scat_wait=False, do_refill=False)
        slot(1, 1)
        slot(2, 2)
        slot(3, 3)

        def quad(j, _):
            c = 4 * j
            for k_ in range(4):
                slot(c + k_, k_)
            return 0

        lax.fori_loop(1, (steps - 6) // 4, quad, 0)

        for q in range(steps - 6, steps):
            k_ = q % 4
            slot(q, k_,
                 do_refill=(q + 3 < steps),
                 do_next=(q + 1 < steps))
        scat_wait((steps - 1) % 4, (steps - 1) % 2)

        if n_extra:
            @pl.when(jnp.logical_and(wid % 8 == 0, wid // 8 < n_extra))
            def _():
                off = nw * steps * _C + (wid // 8) * _C
                pltpu.sync_copy(src_hbm.at[pl.ds(off, _C)], sb[0])
                pltpu.sync_copy(dst_hbm.at[pl.ds(off, _C)], db[0])
                pltpu.async_copy(x_hbm.at[sb[0]], rows[0], ss[0])
                pltpu.make_async_copy(x_hbm.at[sb[0]], rows[0],
                                      ss[0]).wait()
                pltpu.sync_copy(rows[0], accum.at[db[0]], add=True)

        plsc.subcore_barrier()

        pltpu.sync_copy(accum.at[pl.ds(sid * rpt, rpt)],
                        out_hbm.at[cid, pl.ds(sid * rpt, rpt)])
        if rem:
            @pl.when(sid == 0)
            def _():
                pltpu.sync_copy(accum.at[pl.ds(rpt * _NS, rem)],
                                out_hbm.at[cid, pl.ds(rpt * _NS, rem)])

    return k(x, src, dst)


def _tc_dense_body(p0_ref, p1_ref, x_ref, w1l_ref, b1_ref, w1r_ref,
                   w2l_ref, b2_ref, w2r_ref, loc_ref, scale_ref):
    agg = p0_ref[...] + p1_ref[...]
    xb = x_ref[...]
    h1 = (jnp.dot(agg, w1l_ref[...], preferred_element_type=jnp.float32)
          + jnp.dot(xb, w1r_ref[...], preferred_element_type=jnp.float32)
          + b1_ref[...])
    loc_ref[...] = jnp.clip(h1, -100.0, 100.0)
    h2 = (jnp.dot(agg, w2l_ref[...], preferred_element_type=jnp.float32)
          + jnp.dot(xb, w2r_ref[...], preferred_element_type=jnp.float32)
          + b2_ref[...])
    sp = jnp.maximum(h2, 0.0) + jnp.log1p(jnp.exp(-jnp.abs(h2)))
    scale_ref[...] = jnp.minimum(sp + 0.001, 100.0)


def _tc_dense(p0, p1, x, w1lT, b1, w1rT, w2lT, b2, w2rT):
    n, d = x.shape
    blk = 1000
    grid = (n // blk,)
    row_spec = pl.BlockSpec((blk, d), lambda i: (i, 0))
    w_spec = pl.BlockSpec((d, d), lambda i: (0, 0))
    b_spec = pl.BlockSpec((1, d), lambda i: (0, 0))
    return pl.pallas_call(
        _tc_dense_body,
        grid=grid,
        in_specs=[row_spec, row_spec, row_spec,
                  w_spec, b_spec, w_spec, w_spec, b_spec, w_spec],
        out_specs=[row_spec, row_spec],
        out_shape=[jax.ShapeDtypeStruct((n, d), jnp.float32),
                   jax.ShapeDtypeStruct((n, d), jnp.float32)],
    )(p0, p1, x, w1lT, b1, w1rT, w2lT, b2, w2rT)


def kernel(x, edge_index, W1l, b1l, W1r, W2l, b2l, W2r):
    n, d = x.shape
    e = edge_index.shape[1]
    nw = _NC * _NS
    steps = e // (_C * nw)
    steps = steps // 2 * 2
    n_extra = (e - nw * steps * _C) // _C
    if not (e % _C == 0 and n_extra <= 4 and steps % 4 == 2 and steps >= 8):
        raise NotImplementedError("edge count layout not supported")
    src = edge_index[0]
    dst = edge_index[1]
    parts = _sc_agg_parts(n, d, steps, n_extra, x, src, dst)
    loc, scale = _tc_dense(parts[0], parts[1], x,
                           W1l.T, b1l.reshape(1, d), W1r.T,
                           W2l.T, b2l.reshape(1, d), W2r.T)
    return (loc, scale)


# speedup vs baseline: 12.3915x; 1.1079x over previous
import functools

import jax
import jax.numpy as jnp
from jax import lax
from jax.experimental import pallas as pl
from jax.experimental.pallas import tpu as pltpu
from jax.experimental.pallas import tpu_sc as plsc

_NC = 2
_NS = 16
_C = 128


@functools.partial(jax.jit, static_argnums=(0, 1, 2, 3))
def _sc_agg_parts(n, d, steps, n_extra, x, edges):
    e_total = edges.shape[0] // 2
    nw = _NC * _NS
    rpt = (n // _NS) // 8 * 8
    rem = n - rpt * _NS
    full = rpt // _C
    part = rpt - full * _C

    mesh = plsc.VectorSubcoreMesh(core_axis_name="c", subcore_axis_name="s")

    @functools.partial(
        pl.kernel,
        mesh=mesh,
        out_type=[jax.ShapeDtypeStruct((n, d), jnp.float32),
                  jax.ShapeDtypeStruct((n, d), jnp.float32)],
        scratch_types=[
            [pltpu.VMEM((_C,), jnp.int32)] * 4,
            [pltpu.VMEM((_C,), jnp.int32)] * 4,
            [pltpu.VMEM((_C, d), jnp.float32)] * 2,
            [pltpu.SemaphoreType.DMA] * 4,
            [pltpu.SemaphoreType.DMA] * 2,
            pltpu.VMEM_SHARED((n, d), jnp.float32),
        ],
    )
    def k(x_hbm, edges_hbm, out0_hbm, out1_hbm, sb, db, rows, si, ss,
          accum):
        cid = lax.axis_index("c")
        sid = lax.axis_index("s")
        wid = cid * _NS + sid
        ebase = wid * steps * _C

        def idx_start(c, k_):
            off = ebase + c * _C
            pltpu.async_copy(edges_hbm.at[pl.ds(off, _C)], sb[k_], si[k_])
            pltpu.async_copy(edges_hbm.at[pl.ds(e_total + off, _C)],
                             db[k_], si[k_])

        def idx_wait(c, k_):
            off = ebase + c * _C
            pltpu.make_async_copy(edges_hbm.at[pl.ds(off, _C)], sb[k_],
                                  si[k_]).wait()
            pltpu.make_async_copy(edges_hbm.at[pl.ds(e_total + off, _C)],
                                  db[k_], si[k_]).wait()

        def gather_start(k_, r_):
            pltpu.async_copy(x_hbm.at[sb[k_]], rows[r_], ss[r_])

        def gather_wait(k_, r_):
            pltpu.make_async_copy(x_hbm.at[sb[k_]], rows[r_], ss[r_]).wait()

        def scat_start(k_, r_):
            pltpu.async_copy(rows[r_], accum.at[db[k_]], ss[r_], add=True)

        def scat_wait(k_, r_):
            pltpu.make_async_copy(rows[r_], accum.at[db[k_]], ss[r_]).wait()

        for k_ in range(4):
            idx_start(k_, k_)

        zv = jnp.zeros((16,), jnp.float32)

        def zrow(r, _):
            for cc in range(d // 16):
                rows[0][r, pl.ds(cc * 16, 16)] = zv
            return 0

        lax.fori_loop(0, _C, zrow, 0)
        zbase = sid * rpt
        for b in range(full):
            pltpu.sync_copy(rows[0], accum.at[pl.ds(zbase + b * _C, _C)])
        if part:
            pltpu.sync_copy(rows[0].at[pl.ds(0, part)],
                            accum.at[pl.ds(zbase + full * _C, part)])
        if rem:
            @pl.when(sid == 0)
            def _():
                pltpu.sync_copy(rows[0].at[pl.ds(0, rem)],
                                accum.at[pl.ds(rpt * _NS, rem)])

        idx_wait(0, 0)
        gather_start(0, 0)
        plsc.subcore_barrier()

        def slot(q, k_, do_scat_wait=True, do_refill=True, do_next=True):
            if do_next:
                idx_wait(q + 1, (k_ + 1) % 4)
            if do_scat_wait:
                scat_wait((k_ - 1) % 4, (k_ + 1) % 2)
            if do_next:
                gather_start((k_ + 1) % 4, (k_ + 1) % 2)
            if do_refill:
                idx_start(q + 3, (k_ - 1) % 4)
            gather_wait(k_ % 4, k_ % 2)
            scat_start(k_ % 4, k_ % 2)

        slot(0, 0, do_scat_wait=False, do_refill=False)
        slot(1, 1)
        slot(2, 2)
        slot(3, 3)

        def quad(j, _):
            c = 4 * j
            for k_ in range(4):
                slot(c + k_, k_)
            return 0

        lax.fori_loop(1, (steps - 6) // 4, quad, 0)

        for q in range(steps - 6, steps):
            k_ = q % 4
            slot(q, k_,
                 do_refill=(q + 3 < steps),
                 do_next=(q + 1 < steps))
        scat_wait((steps - 1) % 4, (steps - 1) % 2)

        if n_extra:
            @pl.when(jnp.logical_and(wid % 8 == 0, wid // 8 < n_extra))
            def _():
                off = nw * steps * _C + (wid // 8) * _C
                pltpu.sync_copy(edges_hbm.at[pl.ds(off, _C)], sb[0])
                pltpu.sync_copy(edges_hbm.at[pl.ds(e_total + off, _C)],
                                db[0])
                pltpu.async_copy(x_hbm.at[sb[0]], rows[0], ss[0])
                pltpu.make_async_copy(x_hbm.at[sb[0]], rows[0],
                                      ss[0]).wait()
                pltpu.sync_copy(rows[0], accum.at[db[0]], add=True)

        plsc.subcore_barrier()

        @pl.when(cid == 0)
        def _():
            pltpu.sync_copy(accum.at[pl.ds(sid * rpt, rpt)],
                            out0_hbm.at[pl.ds(sid * rpt, rpt)])
            if rem:
                @pl.when(sid == 0)
                def _():
                    pltpu.sync_copy(accum.at[pl.ds(rpt * _NS, rem)],
                                    out0_hbm.at[pl.ds(rpt * _NS, rem)])

        @pl.when(cid == 1)
        def _():
            pltpu.sync_copy(accum.at[pl.ds(sid * rpt, rpt)],
                            out1_hbm.at[pl.ds(sid * rpt, rpt)])
            if rem:
                @pl.when(sid == 0)
                def _():
                    pltpu.sync_copy(accum.at[pl.ds(rpt * _NS, rem)],
                                    out1_hbm.at[pl.ds(rpt * _NS, rem)])

    return k(x, edges)


_DNUM = (((1,), (1,)), ((), ()))


def _tc_xr_body(x_ref, w1r_ref, w2r_ref, xr1_ref, xr2_ref):
    xb = x_ref[...]
    xr1_ref[...] = lax.dot_general(xb, w1r_ref[...], _DNUM,
                                   preferred_element_type=jnp.float32)
    xr2_ref[...] = lax.dot_general(xb, w2r_ref[...], _DNUM,
                                   preferred_element_type=jnp.float32)


def _tc_xr(x, w1r, w2r):
    n, d = x.shape
    blk = 1000
    row_spec = pl.BlockSpec((blk, d), lambda i: (i, 0))
    w_spec = pl.BlockSpec((d, d), lambda i: (0, 0))
    return pl.pallas_call(
        _tc_xr_body,
        grid=(n // blk,),
        in_specs=[row_spec, w_spec, w_spec],
        out_specs=[row_spec, row_spec],
        out_shape=[jax.ShapeDtypeStruct((n, d), jnp.float32),
                   jax.ShapeDtypeStruct((n, d), jnp.float32)],
    )(x, w1r, w2r)


def _tc_final_body(p0_ref, p1_ref, xr1_ref, xr2_ref, w1l_ref, b1_ref,
                   w2l_ref, b2_ref, loc_ref, scale_ref):
    agg = p0_ref[...] + p1_ref[...]
    h1 = (lax.dot_general(agg, w1l_ref[...], _DNUM,
                          preferred_element_type=jnp.float32)
          + xr1_ref[...] + b1_ref[...])
    loc_ref[...] = jnp.clip(h1, -100.0, 100.0)
    h2 = (lax.dot_general(agg, w2l_ref[...], _DNUM,
                          preferred_element_type=jnp.float32)
          + xr2_ref[...] + b2_ref[...])
    sp = jnp.maximum(h2, 0.0) + jnp.log1p(jnp.exp(-jnp.abs(h2)))
    scale_ref[...] = jnp.minimum(sp + 0.001, 100.0)


def _tc_final(p0, p1, xr1, xr2, w1l, b1, w2l, b2):
    n, d = p0.shape
    blk = 1000
    row_spec = pl.BlockSpec((blk, d), lambda i: (i, 0))
    w_spec = pl.BlockSpec((d, d), lambda i: (0, 0))
    b_spec = pl.BlockSpec((1, d), lambda i: (0, 0))
    return pl.pallas_call(
        _tc_final_body,
        grid=(n // blk,),
        in_specs=[row_spec, row_spec, row_spec, row_spec,
                  w_spec, b_spec, w_spec, b_spec],
        out_specs=[row_spec, row_spec],
        out_shape=[jax.ShapeDtypeStruct((n, d), jnp.float32),
                   jax.ShapeDtypeStruct((n, d), jnp.float32)],
    )(p0, p1, xr1, xr2, w1l, b1, w2l, b2)


def kernel(x, edge_index, W1l, b1l, W1r, W2l, b2l, W2r):
    n, d = x.shape
    e = edge_index.shape[1]
    nw = _NC * _NS
    steps = e // (_C * nw)
    steps = steps // 2 * 2
    n_extra = (e - nw * steps * _C) // _C
    if not (e % _C == 0 and n_extra <= 4 and steps % 4 == 2 and steps >= 8):
        raise NotImplementedError("edge count layout not supported")
    edges = edge_index.reshape(2 * e)
    p0, p1 = _sc_agg_parts(n, d, steps, n_extra, x, edges)
    xr1, xr2 = _tc_xr(x, W1r, W2r)
    loc, scale = _tc_final(p0, p1, xr1, xr2,
                           W1l, b1l.reshape(1, d), W2l, b2l.reshape(1, d))
    return (loc, scale)


# speedup vs baseline: 12.7761x; 1.0310x over previous
import functools

import jax
import jax.numpy as jnp
from jax import lax
from jax.experimental import pallas as pl
from jax.experimental.pallas import tpu as pltpu
from jax.experimental.pallas import tpu_sc as plsc

_NC = 2
_NS = 16
_C = 128


@functools.partial(jax.jit, static_argnums=(0, 1, 2, 3))
def _sc_agg_parts(n, d, steps, n_extra, x, edges):
    nw = _NC * _NS
    rpt = (n // _NS) // 8 * 8
    rem = n - rpt * _NS
    full = rpt // _C
    part = rpt - full * _C

    mesh = plsc.VectorSubcoreMesh(core_axis_name="c", subcore_axis_name="s")

    @functools.partial(
        pl.kernel,
        mesh=mesh,
        out_type=[jax.ShapeDtypeStruct((n, d), jnp.float32),
                  jax.ShapeDtypeStruct((n, d), jnp.float32)],
        scratch_types=[
            [pltpu.VMEM((2, _C), jnp.int32)] * 4,
            [pltpu.VMEM((_C, d), jnp.float32)] * 2,
            [pltpu.SemaphoreType.DMA] * 4,
            [pltpu.SemaphoreType.DMA] * 2,
            pltpu.VMEM_SHARED((n, d), jnp.float32),
        ],
    )
    def k(x_hbm, edges_hbm, out0_hbm, out1_hbm, eb, rows, si, ss,
          accum):
        cid = lax.axis_index("c")
        sid = lax.axis_index("s")
        wid = cid * _NS + sid
        ebase = wid * steps * _C

        def idx_start(c, k_):
            off = ebase + c * _C
            pltpu.async_copy(edges_hbm.at[:, pl.ds(off, _C)], eb[k_],
                             si[k_])

        def idx_wait(c, k_):
            off = ebase + c * _C
            pltpu.make_async_copy(edges_hbm.at[:, pl.ds(off, _C)], eb[k_],
                                  si[k_]).wait()

        def gather_start(k_, r_):
            pltpu.async_copy(x_hbm.at[eb[k_].at[0]], rows[r_], ss[r_])

        def gather_wait(k_, r_):
            pltpu.make_async_copy(x_hbm.at[eb[k_].at[0]], rows[r_],
                                  ss[r_]).wait()

        def scat_start(k_, r_):
            pltpu.async_copy(rows[r_], accum.at[eb[k_].at[1]], ss[r_],
                             add=True)

        def scat_wait(k_, r_):
            pltpu.make_async_copy(rows[r_], accum.at[eb[k_].at[1]],
                                  ss[r_]).wait()

        for k_ in range(4):
            idx_start(k_, k_)

        zv = jnp.zeros((16,), jnp.float32)

        def zrow(r, _):
            for cc in range(d // 16):
                rows[0][r, pl.ds(cc * 16, 16)] = zv
            return 0

        lax.fori_loop(0, _C, zrow, 0)
        zbase = sid * rpt
        for b in range(full):
            pltpu.sync_copy(rows[0], accum.at[pl.ds(zbase + b * _C, _C)])
        if part:
            pltpu.sync_copy(rows[0].at[pl.ds(0, part)],
                            accum.at[pl.ds(zbase + full * _C, part)])
        if rem:
            @pl.when(sid == 0)
            def _():
                pltpu.sync_copy(rows[0].at[pl.ds(0, rem)],
                                accum.at[pl.ds(rpt * _NS, rem)])

        idx_wait(0, 0)
        gather_start(0, 0)
        plsc.subcore_barrier()

        def slot(q, k_, do_scat_wait=True, do_refill=True, do_next=True):
            if do_next:
                idx_wait(q + 1, (k_ + 1) % 4)
            if do_scat_wait:
                scat_wait((k_ - 1) % 4, (k_ + 1) % 2)
            if do_next:
                gather_start((k_ + 1) % 4, (k_ + 1) % 2)
            if do_refill:
                idx_start(q + 3, (k_ - 1) % 4)
            gather_wait(k_ % 4, k_ % 2)
            scat_start(k_ % 4, k_ % 2)

        slot(0, 0, do_scat_wait=False, do_refill=False)
        slot(1, 1)
        slot(2, 2)
        slot(3, 3)

        def quad(j, _):
            c = 4 * j
            for k_ in range(4):
                slot(c + k_, k_)
            return 0

        lax.fori_loop(1, (steps - 6) // 4, quad, 0)

        for q in range(steps - 6, steps):
            k_ = q % 4
            slot(q, k_,
                 do_refill=(q + 3 < steps),
                 do_next=(q + 1 < steps))
        scat_wait((steps - 1) % 4, (steps - 1) % 2)

        if n_extra:
            @pl.when(jnp.logical_and(wid % 8 == 0, wid // 8 < n_extra))
            def _():
                off = nw * steps * _C + (wid // 8) * _C
                pltpu.sync_copy(edges_hbm.at[:, pl.ds(off, _C)], eb[0])
                pltpu.async_copy(x_hbm.at[eb[0].at[0]], rows[0], ss[0])
                pltpu.make_async_copy(x_hbm.at[eb[0].at[0]], rows[0],
                                      ss[0]).wait()
                pltpu.sync_copy(rows[0], accum.at[eb[0].at[1]], add=True)

        plsc.subcore_barrier()

        @pl.when(cid == 0)
        def _():
            pltpu.sync_copy(accum.at[pl.ds(sid * rpt, rpt)],
                            out0_hbm.at[pl.ds(sid * rpt, rpt)])
            if rem:
                @pl.when(sid == 0)
                def _():
                    pltpu.sync_copy(accum.at[pl.ds(rpt * _NS, rem)],
                                    out0_hbm.at[pl.ds(rpt * _NS, rem)])

        @pl.when(cid == 1)
        def _():
            pltpu.sync_copy(accum.at[pl.ds(sid * rpt, rpt)],
                            out1_hbm.at[pl.ds(sid * rpt, rpt)])
            if rem:
                @pl.when(sid == 0)
                def _():
                    pltpu.sync_copy(accum.at[pl.ds(rpt * _NS, rem)],
                                    out1_hbm.at[pl.ds(rpt * _NS, rem)])

    return k(x, edges)


_DNUM = (((1,), (1,)), ((), ()))


def _tc_final_body(p0_ref, p1_ref, x_ref, w1l_ref, b1_ref, w1r_ref,
                   w2l_ref, b2_ref, w2r_ref, loc_ref, scale_ref):
    agg = p0_ref[...] + p1_ref[...]
    xb = x_ref[...]
    h1 = (lax.dot_general(agg, w1l_ref[...], _DNUM,
                          preferred_element_type=jnp.float32)
          + lax.dot_general(xb, w1r_ref[...], _DNUM,
                            preferred_element_type=jnp.float32)
          + b1_ref[...])
    loc_ref[...] = jnp.clip(h1, -100.0, 100.0)
    h2 = (lax.dot_general(agg, w2l_ref[...], _DNUM,
                          preferred_element_type=jnp.float32)
          + lax.dot_general(xb, w2r_ref[...], _DNUM,
                            preferred_element_type=jnp.float32)
          + b2_ref[...])
    sp = jnp.maximum(h2, 0.0) + jnp.log1p(jnp.exp(-jnp.abs(h2)))
    scale_ref[...] = jnp.minimum(sp + 0.001, 100.0)


def _tc_final(p0, p1, x, w1l, b1, w1r, w2l, b2, w2r):
    n, d = p0.shape
    blk = 1000
    row_spec = pl.BlockSpec((blk, d), lambda i: (i, 0))
    w_spec = pl.BlockSpec((d, d), lambda i: (0, 0))
    b_spec = pl.BlockSpec((1, d), lambda i: (0, 0))
    return pl.pallas_call(
        _tc_final_body,
        grid=(n // blk,),
        in_specs=[row_spec, row_spec, row_spec,
                  w_spec, b_spec, w_spec, w_spec, b_spec, w_spec],
        out_specs=[row_spec, row_spec],
        out_shape=[jax.ShapeDtypeStruct((n, d), jnp.float32),
                   jax.ShapeDtypeStruct((n, d), jnp.float32)],
    )(p0, p1, x, w1l, b1, w1r, w2l, b2, w2r)


def kernel(x, edge_index, W1l, b1l, W1r, W2l, b2l, W2r):
    n, d = x.shape
    e = edge_index.shape[1]
    nw = _NC * _NS
    steps = e // (_C * nw)
    steps = steps // 2 * 2
    n_extra = (e - nw * steps * _C) // _C
    if not (e % _C == 0 and n_extra <= 4 and steps % 4 == 2 and steps >= 8):
        raise NotImplementedError("edge count layout not supported")
    p0, p1 = _sc_agg_parts(n, d, steps, n_extra, x, edge_index)
    loc, scale = _tc_final(p0, p1, x, W1l, b1l.reshape(1, d), W1r,
                           W2l, b2l.reshape(1, d), W2r)
    return (loc, scale)


# speedup vs baseline: 13.1289x; 1.0276x over previous
import functools

import jax
import jax.numpy as jnp
from jax import lax
from jax.experimental import pallas as pl
from jax.experimental.pallas import tpu as pltpu
from jax.experimental.pallas import tpu_sc as plsc

_NC = 2
_NS = 16
_C = 128


@functools.partial(jax.jit, static_argnums=(0, 1, 2, 3))
def _sc_agg_parts(n, d, steps, n_extra, x, edges):
    nw = _NC * _NS
    rpt = (n // _NS) // 8 * 8
    rem = n - rpt * _NS
    full = rpt // _C
    part = rpt - full * _C

    mesh = plsc.VectorSubcoreMesh(core_axis_name="c", subcore_axis_name="s")

    @functools.partial(
        pl.kernel,
        mesh=mesh,
        out_type=[jax.ShapeDtypeStruct((n, d), jnp.float32),
                  jax.ShapeDtypeStruct((n, d), jnp.float32)],
        scratch_types=[
            [pltpu.VMEM((2, _C), jnp.int32)] * 4,
            [pltpu.VMEM((_C, d), jnp.float32)] * 2,
            [pltpu.SemaphoreType.DMA] * 4,
            [pltpu.SemaphoreType.DMA] * 2,
            pltpu.VMEM_SHARED((n, d), jnp.float32),
        ],
    )
    def k(x_hbm, edges_hbm, out0_hbm, out1_hbm, eb, rows, si, ss,
          accum):
        cid = lax.axis_index("c")
        sid = lax.axis_index("s")
        wid = cid * _NS + sid
        ebase = wid * steps * _C

        def idx_start(c, k_):
            off = ebase + c * _C
            pltpu.async_copy(edges_hbm.at[:, pl.ds(off, _C)], eb[k_],
                             si[k_])

        def idx_wait(c, k_):
            off = ebase + c * _C
            pltpu.make_async_copy(edges_hbm.at[:, pl.ds(off, _C)], eb[k_],
                                  si[k_]).wait()

        def gather_start(k_, r_):
            pltpu.async_copy(x_hbm.at[eb[k_].at[0]], rows[r_], ss[r_])

        def gather_wait(k_, r_):
            pltpu.make_async_copy(x_hbm.at[eb[k_].at[0]], rows[r_],
                                  ss[r_]).wait()

        def scat_start(k_, r_):
            pltpu.async_copy(rows[r_], accum.at[eb[k_].at[1]], ss[r_],
                             add=True)

        def scat_wait(k_, r_):
            pltpu.make_async_copy(rows[r_], accum.at[eb[k_].at[1]],
                                  ss[r_]).wait()

        for k_ in range(4):
            idx_start(k_, k_)
        idx_wait(0, 0)
        gather_start(0, 0)

        zv = jnp.zeros((16,), jnp.float32)

        def zrow(r, _):
            for cc in range(d // 16):
                rows[1][r, pl.ds(cc * 16, 16)] = zv
            return 0

        lax.fori_loop(0, _C, zrow, 0)
        zbase = sid * rpt
        for b in range(full):
            pltpu.sync_copy(rows[1], accum.at[pl.ds(zbase + b * _C, _C)])
        if part:
            pltpu.sync_copy(rows[1].at[pl.ds(0, part)],
                            accum.at[pl.ds(zbase + full * _C, part)])
        if rem:
            @pl.when(sid == 0)
            def _():
                pltpu.sync_copy(rows[1].at[pl.ds(0, rem)],
                                accum.at[pl.ds(rpt * _NS, rem)])

        plsc.subcore_barrier()

        def slot(q, k_, do_scat_wait=True, do_refill=True, do_next=True):
            if do_next:
                idx_wait(q + 1, (k_ + 1) % 4)
            if do_scat_wait:
                scat_wait((k_ - 1) % 4, (k_ + 1) % 2)
            if do_next:
                gather_start((k_ + 1) % 4, (k_ + 1) % 2)
            if do_refill:
                idx_start(q + 3, (k_ - 1) % 4)
            gather_wait(k_ % 4, k_ % 2)
            scat_start(k_ % 4, k_ % 2)

        slot(0, 0, do_scat_wait=False, do_refill=False)
        slot(1, 1)
        slot(2, 2)
        slot(3, 3)

        def quad(j, _):
            c = 4 * j
            for k_ in range(4):
                slot(c + k_, k_)
            return 0

        lax.fori_loop(1, (steps - 6) // 4, quad, 0)

        for q in range(steps - 6, steps):
            k_ = q % 4
            slot(q, k_,
                 do_refill=(q + 3 < steps),
                 do_next=(q + 1 < steps))
        scat_wait((steps - 1) % 4, (steps - 1) % 2)

        if n_extra:
            @pl.when(jnp.logical_and(wid % 8 == 0, wid // 8 < n_extra))
            def _():
                off = nw * steps * _C + (wid // 8) * _C
                pltpu.sync_copy(edges_hbm.at[:, pl.ds(off, _C)], eb[0])
                pltpu.async_copy(x_hbm.at[eb[0].at[0]], rows[0], ss[0])
                pltpu.make_async_copy(x_hbm.at[eb[0].at[0]], rows[0],
                                      ss[0]).wait()
                pltpu.sync_copy(rows[0], accum.at[eb[0].at[1]], add=True)

        plsc.subcore_barrier()

        @pl.when(cid == 0)
        def _():
            pltpu.sync_copy(accum.at[pl.ds(sid * rpt, rpt)],
                            out0_hbm.at[pl.ds(sid * rpt, rpt)])
            if rem:
                @pl.when(sid == 0)
                def _():
                    pltpu.sync_copy(accum.at[pl.ds(rpt * _NS, rem)],
                                    out0_hbm.at[pl.ds(rpt * _NS, rem)])

        @pl.when(cid == 1)
        def _():
            pltpu.sync_copy(accum.at[pl.ds(sid * rpt, rpt)],
                            out1_hbm.at[pl.ds(sid * rpt, rpt)])
            if rem:
                @pl.when(sid == 0)
                def _():
                    pltpu.sync_copy(accum.at[pl.ds(rpt * _NS, rem)],
                                    out1_hbm.at[pl.ds(rpt * _NS, rem)])

    return k(x, edges)


_DNUM = (((1,), (1,)), ((), ()))


def _tc_final_body(p0_ref, p1_ref, x_ref, w1l_ref, b1_ref, w1r_ref,
                   w2l_ref, b2_ref, w2r_ref, loc_ref, scale_ref):
    agg = p0_ref[...] + p1_ref[...]
    xb = x_ref[...]
    h1 = (lax.dot_general(agg, w1l_ref[...], _DNUM,
                          preferred_element_type=jnp.float32)
          + lax.dot_general(xb, w1r_ref[...], _DNUM,
                            preferred_element_type=jnp.float32)
          + b1_ref[...])
    loc_ref[...] = jnp.clip(h1, -100.0, 100.0)
    h2 = (lax.dot_general(agg, w2l_ref[...], _DNUM,
                          preferred_element_type=jnp.float32)
          + lax.dot_general(xb, w2r_ref[...], _DNUM,
                            preferred_element_type=jnp.float32)
          + b2_ref[...])
    sp = jnp.maximum(h2, 0.0) + jnp.log1p(jnp.exp(-jnp.abs(h2)))
    scale_ref[...] = jnp.minimum(sp + 0.001, 100.0)


def _tc_final(p0, p1, x, w1l, b1, w1r, w2l, b2, w2r):
    n, d = p0.shape
    blk = 2000
    row_spec = pl.BlockSpec((blk, d), lambda i: (i, 0))
    w_spec = pl.BlockSpec((d, d), lambda i: (0, 0))
    b_spec = pl.BlockSpec((1, d), lambda i: (0, 0))
    return pl.pallas_call(
        _tc_final_body,
        grid=(n // blk,),
        in_specs=[row_spec, row_spec, row_spec,
                  w_spec, b_spec, w_spec, w_spec, b_spec, w_spec],
        out_specs=[row_spec, row_spec],
        out_shape=[jax.ShapeDtypeStruct((n, d), jnp.float32),
                   jax.ShapeDtypeStruct((n, d), jnp.float32)],
    )(p0, p1, x, w1l, b1, w1r, w2l, b2, w2r)


def kernel(x, edge_index, W1l, b1l, W1r, W2l, b2l, W2r):
    n, d = x.shape
    e = edge_index.shape[1]
    nw = _NC * _NS
    steps = e // (_C * nw)
    steps = steps // 2 * 2
    n_extra = (e - nw * steps * _C) // _C
    if not (e % _C == 0 and n_extra <= 4 and steps % 4 == 2 and steps >= 8):
        raise NotImplementedError("edge count layout not supported")
    p0, p1 = _sc_agg_parts(n, d, steps, n_extra, x, edge_index)
    loc, scale = _tc_final(p0, p1, x, W1l, b1l.reshape(1, d), W1r,
                           W2l, b2l.reshape(1, d), W2r)
    return (loc, scale)
